# baseline (speedup 1.0000x reference)
import functools

import jax
import jax.numpy as jnp
import numpy as np
from jax import lax
from jax.experimental import pallas as pl
from jax.experimental.pallas import tpu as pltpu
from jax.experimental.pallas import tpu_sc as plsc

L, B, D = 200, 256, 64
R = L * B
K = 1000
CHUNK = 128
NODE_CHUNKS = R // CHUNK
CPW = NODE_CHUNKS // 16
MM_LS = 4
FIX_LS = 8

_SEL = np.zeros((CHUNK, B), dtype=np.float32)
_SEL[np.arange(B) // 2, np.arange(B)] = 1.0


def _pe_np():
    pos = np.arange(L, dtype=np.float32)[:, None]
    div = np.exp(np.arange(0, D, 2, dtype=np.float32) * (-np.log(10000.0) / D))
    pe = np.zeros((L, D), dtype=np.float32)
    pe[:, 0::2] = np.sin(pos * div)
    pe[:, 1::2] = np.cos(pos * div)
    return pe


_PE3 = _pe_np()[:, :, None]


def _sc_gather(node_idx_h, edge_idx_h, node_tab_h, edge_tab_h,
               out_h, idx_v, rows_v, sem):
    cid = lax.axis_index("c")
    sid = lax.axis_index("s")
    wid = sid * 2 + cid

    def do_chunks(idx_h, tab_h, w, out_row0):
        def body(k, _):
            c = w * CPW + k
            pltpu.sync_copy(idx_h.at[pl.ds(c * CHUNK, CHUNK)], idx_v)
            pltpu.async_copy(tab_h.at[idx_v], rows_v, sem).wait()
            lb = c // 2
            h = c % 2
            pltpu.sync_copy(
                rows_v,
                out_h.at[pl.ds(out_row0 + lb * CHUNK, CHUNK),
                         pl.ds(h * D, D)],
            )
            return 0

        lax.fori_loop(0, CPW, body, 0)

    @pl.when(wid < 16)
    def _():
        do_chunks(node_idx_h, node_tab_h, wid, 0)

    @pl.when(wid >= 16)
    def _():
        do_chunks(edge_idx_h, edge_tab_h, wid - 16, R // 2)


def _make_sc_call():
    mesh = plsc.VectorSubcoreMesh(core_axis_name="c", subcore_axis_name="s")
    return pl.kernel(
        _sc_gather,
        out_type=jax.ShapeDtypeStruct((R, 2 * D), jnp.float32),
        mesh=mesh,
        compiler_params=pltpu.CompilerParams(use_tc_tiling_on_sc=False),
        scratch_types=[
            pltpu.VMEM((CHUNK,), jnp.int32),
            pltpu.VMEM((CHUNK, D), jnp.float32),
            pltpu.SemaphoreType.DMA,
        ],
    )


def _mm_body(xt_ref, wt_ref, pet_ref, out_ref):
    prod = jnp.dot(wt_ref[...], xt_ref[...], preferred_element_type=jnp.float32)
    for j in range(MM_LS):
        out_ref[j] = prod[:, j * B:(j + 1) * B] + pet_ref[j]


def _mm_call(xt, wt, pet):
    return pl.pallas_call(
        _mm_body,
        grid=(L // MM_LS,),
        in_specs=[
            pl.BlockSpec((K, MM_LS * B), lambda i: (0, i)),
            pl.BlockSpec((D, K), lambda i: (0, 0)),
            pl.BlockSpec((MM_LS, D, 1), lambda i: (i, 0, 0)),
        ],
        out_specs=pl.BlockSpec((MM_LS, D, B), lambda i: (2 * L // MM_LS + i, 0, 0)),
        out_shape=jax.ShapeDtypeStruct((3 * L, D, B), jnp.float32),
    )(xt, wt, pet)


def _fix_body(alias_ref, tmp_ref, pet_ref, out_ref):
    del alias_ref
    for j in range(FIX_LS):
        t = tmp_ref[pl.ds(j * CHUNK, CHUNK), :]
        pe = pet_ref[j]
        out_ref[j, :, :CHUNK] = t[:, :D].T + pe
        out_ref[j, :, CHUNK:] = t[:, D:].T + pe


def _fix_call(out_a, tmp, pet):
    n_pe_blocks = L // FIX_LS
    return pl.pallas_call(
        _fix_body,
        grid=(2 * L // FIX_LS,),
        in_specs=[
            pl.BlockSpec(memory_space=pl.ANY),
            pl.BlockSpec((FIX_LS * CHUNK, 2 * D), lambda i: (i, 0)),
            pl.BlockSpec((FIX_LS, D, 1),
                         lambda i: (lax.rem(i, n_pe_blocks), 0, 0)),
        ],
        out_specs=pl.BlockSpec((FIX_LS, D, B), lambda i: (i, 0, 0)),
        out_shape=jax.ShapeDtypeStruct((3 * L, D, B), jnp.float32),
        input_output_aliases={0: 0},
    )(out_a, tmp, pet)


def kernel(node_idx, edge_idx, node_val_mat, node_embed_table,
           edge_embed_table, val_tok_embed):
    pet = jnp.asarray(_PE3)
    xt = jnp.transpose(node_val_mat)
    wt = jnp.transpose(val_tok_embed)
    tmp = _make_sc_call()(
        node_idx.reshape(-1).astype(jnp.int32),
        edge_idx.reshape(-1).astype(jnp.int32),
        node_embed_table,
        edge_embed_table,
    )
    out_a = _mm_call(xt, wt, pet)
    out = _fix_call(out_a, tmp, pet)
    return jnp.transpose(out.reshape(3 * L, D, B), (0, 2, 1))


# speedup vs baseline: 1.0585x; 1.0585x over previous
import functools

import jax
import jax.numpy as jnp
import numpy as np
from jax import lax
from jax.experimental import pallas as pl
from jax.experimental.pallas import tpu as pltpu
from jax.experimental.pallas import tpu_sc as plsc

L, B, D = 200, 256, 64
R = L * B
K = 1000
CHUNK = 128
NODE_CHUNKS = R // CHUNK
CPW = NODE_CHUNKS // 16
MM_LS = 8
FIX_LS = 8

_SEL = np.zeros((CHUNK, B), dtype=np.float32)
_SEL[np.arange(B) // 2, np.arange(B)] = 1.0


def _pe_np():
    pos = np.arange(L, dtype=np.float32)[:, None]
    div = np.exp(np.arange(0, D, 2, dtype=np.float32) * (-np.log(10000.0) / D))
    pe = np.zeros((L, D), dtype=np.float32)
    pe[:, 0::2] = np.sin(pos * div)
    pe[:, 1::2] = np.cos(pos * div)
    return pe


_PE3 = _pe_np()[:, :, None]


def _sc_gather(node_idx_h, edge_idx_h, node_tab_h, edge_tab_h,
               out_h, idx_v, rows_v, sem):
    cid = lax.axis_index("c")
    sid = lax.axis_index("s")
    wid = sid * 2 + cid

    def do_chunks(idx_h, tab_h, w, out_row0):
        def body(k, _):
            c = w * CPW + k
            pltpu.sync_copy(idx_h.at[pl.ds(c * CHUNK, CHUNK)], idx_v)
            pltpu.async_copy(tab_h.at[idx_v], rows_v, sem).wait()
            pltpu.sync_copy(
                rows_v,
                out_h.at[pl.ds(out_row0 + c * CHUNK, CHUNK)],
            )
            return 0

        lax.fori_loop(0, CPW, body, 0)

    @pl.when(wid < 16)
    def _():
        do_chunks(node_idx_h, node_tab_h, wid, 0)

    @pl.when(wid >= 16)
    def _():
        do_chunks(edge_idx_h, edge_tab_h, wid - 16, R)


def _make_sc_call():
    mesh = plsc.VectorSubcoreMesh(core_axis_name="c", subcore_axis_name="s")
    return pl.kernel(
        _sc_gather,
        out_type=jax.ShapeDtypeStruct((2 * R, D), jnp.float32),
        mesh=mesh,
        compiler_params=pltpu.CompilerParams(use_tc_tiling_on_sc=False),
        scratch_types=[
            pltpu.VMEM((CHUNK,), jnp.int32),
            pltpu.VMEM((CHUNK, D), jnp.float32),
            pltpu.SemaphoreType.DMA,
        ],
    )


def _mm_body(xt_ref, wt_ref, pet_ref, out_ref):
    prod = jnp.dot(wt_ref[...], xt_ref[...], preferred_element_type=jnp.float32)
    for j in range(MM_LS):
        out_ref[j] = prod[:, j * B:(j + 1) * B] + pet_ref[j]


def _mm_call(xt, wt, pet):
    return pl.pallas_call(
        _mm_body,
        grid=(L // MM_LS,),
        in_specs=[
            pl.BlockSpec((K, MM_LS * B), lambda i: (0, i)),
            pl.BlockSpec((D, K), lambda i: (0, 0)),
            pl.BlockSpec((MM_LS, D, 1), lambda i: (i, 0, 0)),
        ],
        out_specs=pl.BlockSpec((MM_LS, D, B), lambda i: (2 * L // MM_LS + i, 0, 0)),
        out_shape=jax.ShapeDtypeStruct((3 * L, D, B), jnp.float32),
    )(xt, wt, pet)


def _fix_body(alias_ref, tmp_ref, sel_ref, pet_ref, out_ref):
    del alias_ref
    b_i = lax.broadcasted_iota(jnp.int32, (D, B), 1)
    for j in range(FIX_LS):
        t = tmp_ref[pl.ds(j * CHUNK, CHUNK), :]
        h = lax.dot_general(t, sel_ref[...], (((0,), (0,)), ((), ())),
                            preferred_element_type=jnp.float32)
        out = jnp.where(b_i % 2 == 0, h[:D, :], h[D:, :])
        out_ref[j] = out + pet_ref[j]


def _fix_call(out_a, tmp, sel, pet):
    n_pe_blocks = L // FIX_LS
    return pl.pallas_call(
        _fix_body,
        grid=(2 * L // FIX_LS,),
        in_specs=[
            pl.BlockSpec(memory_space=pl.ANY),
            pl.BlockSpec((FIX_LS * CHUNK, 2 * D), lambda i: (i, 0)),
            pl.BlockSpec((CHUNK, B), lambda i: (0, 0)),
            pl.BlockSpec((FIX_LS, D, 1),
                         lambda i: (lax.rem(i, n_pe_blocks), 0, 0)),
        ],
        out_specs=pl.BlockSpec((FIX_LS, D, B), lambda i: (i, 0, 0)),
        out_shape=jax.ShapeDtypeStruct((3 * L, D, B), jnp.float32),
        input_output_aliases={0: 0},
    )(out_a, tmp, sel, pet)


def kernel(node_idx, edge_idx, node_val_mat, node_embed_table,
           edge_embed_table, val_tok_embed):
    pet = jnp.asarray(_PE3)
    sel = jnp.asarray(_SEL)
    xt = jnp.transpose(node_val_mat)
    wt = jnp.transpose(val_tok_embed)
    nt_lin = lax.optimization_barrier(node_embed_table.reshape(-1))
    tmp = _make_sc_call()(
        node_idx.reshape(-1).astype(jnp.int32),
        edge_idx.reshape(-1).astype(jnp.int32),
        nt_lin.reshape(100000, D),
        edge_embed_table,
    ).reshape(R, 2 * D)
    out_a = _mm_call(xt, wt, pet)
    out = _fix_call(out_a, tmp, sel, pet)
    return jnp.transpose(out.reshape(3 * L, D, B), (0, 2, 1))


# speedup vs baseline: 1.0853x; 1.0253x over previous
import functools

import jax
import jax.numpy as jnp
import numpy as np
from jax import lax
from jax.experimental import pallas as pl
from jax.experimental.pallas import tpu as pltpu
from jax.experimental.pallas import tpu_sc as plsc

L, B, D = 200, 256, 64
R = L * B
K = 1000
CHUNK = 128
NODE_CHUNKS = R // CHUNK
CPW = NODE_CHUNKS // 16
MM_LS = 8
FIX_LS = 10

_SEL = np.zeros((CHUNK, B), dtype=np.float32)
_SEL[np.arange(B) // 2, np.arange(B)] = 1.0


def _pe_np():
    pos = np.arange(L, dtype=np.float32)[:, None]
    div = np.exp(np.arange(0, D, 2, dtype=np.float32) * (-np.log(10000.0) / D))
    pe = np.zeros((L, D), dtype=np.float32)
    pe[:, 0::2] = np.sin(pos * div)
    pe[:, 1::2] = np.cos(pos * div)
    return pe


_PE3 = _pe_np()[:, :, None]


def _sc_gather(node_idx_h, edge_idx_h, node_tab_h, edge_tab_h,
               out_h, idx_v, rows_v, sem):
    cid = lax.axis_index("c")
    sid = lax.axis_index("s")
    wid = sid * 2 + cid

    def do_chunks(idx_h, tab_h, w, out_row0):
        def body(k, _):
            c = w * CPW + k
            pltpu.sync_copy(idx_h.at[pl.ds(c * CHUNK, CHUNK)], idx_v)
            pltpu.async_copy(tab_h.at[idx_v], rows_v, sem).wait()
            pltpu.sync_copy(
                rows_v,
                out_h.at[pl.ds(out_row0 + c * CHUNK, CHUNK)],
            )
            return 0

        lax.fori_loop(0, CPW, body, 0)

    @pl.when(wid < 16)
    def _():
        do_chunks(node_idx_h, node_tab_h, wid, 0)

    @pl.when(wid >= 16)
    def _():
        do_chunks(edge_idx_h, edge_tab_h, wid - 16, R)


def _make_sc_call():
    mesh = plsc.VectorSubcoreMesh(core_axis_name="c", subcore_axis_name="s")
    return pl.kernel(
        _sc_gather,
        out_type=jax.ShapeDtypeStruct((2 * R, D), jnp.float32),
        mesh=mesh,
        compiler_params=pltpu.CompilerParams(use_tc_tiling_on_sc=False),
        scratch_types=[
            pltpu.VMEM((CHUNK,), jnp.int32),
            pltpu.VMEM((CHUNK, D), jnp.float32),
            pltpu.SemaphoreType.DMA,
        ],
    )


def _mm_body(xt_ref, wt_ref, pet_ref, out_ref):
    prod = jnp.dot(wt_ref[...], xt_ref[...], preferred_element_type=jnp.float32)
    for j in range(MM_LS):
        out_ref[j] = prod[:, j * B:(j + 1) * B] + pet_ref[j]


def _mm_call(xt, wt, pet):
    return pl.pallas_call(
        _mm_body,
        grid=(L // MM_LS,),
        in_specs=[
            pl.BlockSpec((K, MM_LS * B), lambda i: (0, i)),
            pl.BlockSpec((D, K), lambda i: (0, 0)),
            pl.BlockSpec((MM_LS, D, 1), lambda i: (i, 0, 0)),
        ],
        out_specs=pl.BlockSpec((MM_LS, D, B), lambda i: (2 * L // MM_LS + i, 0, 0)),
        out_shape=jax.ShapeDtypeStruct((3 * L, D, B), jnp.float32),
    )(xt, wt, pet)


def _fix_body(alias_ref, tmp_ref, sel_ref, pet_ref, out_ref):
    del alias_ref
    b_i = lax.broadcasted_iota(jnp.int32, (D, B), 1)
    for j in range(FIX_LS):
        t = tmp_ref[pl.ds(j * CHUNK, CHUNK), :]
        h = lax.dot_general(t, sel_ref[...], (((0,), (0,)), ((), ())),
                            preferred_element_type=jnp.float32)
        out = jnp.where(b_i % 2 == 0, h[:D, :], h[D:, :])
        out_ref[j] = out + pet_ref[j]


def _fix_call(out_a, tmp, sel, pet):
    n_pe_blocks = L // FIX_LS
    return pl.pallas_call(
        _fix_body,
        grid=(2 * L // FIX_LS,),
        in_specs=[
            pl.BlockSpec(memory_space=pl.ANY),
            pl.BlockSpec((FIX_LS * CHUNK, 2 * D), lambda i: (i, 0)),
            pl.BlockSpec((CHUNK, B), lambda i: (0, 0)),
            pl.BlockSpec((FIX_LS, D, 1),
                         lambda i: (lax.rem(i, n_pe_blocks), 0, 0)),
        ],
        out_specs=pl.BlockSpec((FIX_LS, D, B), lambda i: (i, 0, 0)),
        out_shape=jax.ShapeDtypeStruct((3 * L, D, B), jnp.float32),
        input_output_aliases={0: 0},
    )(out_a, tmp, sel, pet)


def kernel(node_idx, edge_idx, node_val_mat, node_embed_table,
           edge_embed_table, val_tok_embed):
    pet = jnp.asarray(_PE3)
    sel = jnp.asarray(_SEL)
    xt = jnp.transpose(node_val_mat)
    wt = jnp.transpose(val_tok_embed)
    nt_lin = lax.optimization_barrier(node_embed_table.reshape(-1))
    tmp = _make_sc_call()(
        node_idx.reshape(-1).astype(jnp.int32),
        edge_idx.reshape(-1).astype(jnp.int32),
        nt_lin.reshape(100000, D),
        edge_embed_table,
    ).reshape(R, 2 * D)
    out_a = _mm_call(xt, wt, pet)
    out = _fix_call(out_a, tmp, sel, pet)
    return jnp.transpose(out.reshape(3 * L, D, B), (0, 2, 1))


# speedup vs baseline: 1.1440x; 1.0541x over previous
import jax
import jax.numpy as jnp
import numpy as np
from jax import lax
from jax.experimental import pallas as pl
from jax.experimental.pallas import tpu as pltpu
from jax.experimental.pallas import tpu_sc as plsc

L, B, D = 200, 256, 64
R = L * B
K = 1000
CHUNK = 128
NODE_CHUNKS = R // CHUNK
CPW = NODE_CHUNKS // 16
MM_LS = 10
FIX_LS = 20

_SEL = np.zeros((CHUNK, B), dtype=np.float32)
_SEL[np.arange(B) // 2, np.arange(B)] = 1.0


def _pe_np():
    pos = np.arange(L, dtype=np.float32)[:, None]
    div = np.exp(np.arange(0, D, 2, dtype=np.float32) * (-np.log(10000.0) / D))
    pe = np.zeros((L, D), dtype=np.float32)
    pe[:, 0::2] = np.sin(pos * div)
    pe[:, 1::2] = np.cos(pos * div)
    return pe


_PE3 = _pe_np()[:, :, None]


def _sc_gather(node_idx_h, edge_idx_h, node_tab_h, edge_tab_h,
               out_h, idx_v, rows_v, sem):
    cid = lax.axis_index("c")
    sid = lax.axis_index("s")
    wid = sid * 2 + cid

    def do_chunks(idx_h, tab_h, w, out_row0):
        def body(k, _):
            c = w * CPW + k
            pltpu.sync_copy(idx_h.at[pl.ds(c * CHUNK, CHUNK)], idx_v)
            pltpu.async_copy(tab_h.at[idx_v], rows_v, sem).wait()
            pltpu.sync_copy(
                rows_v,
                out_h.at[pl.ds(out_row0 + c * CHUNK, CHUNK)],
            )
            return 0

        lax.fori_loop(0, CPW, body, 0)

    @pl.when(wid < 16)
    def _():
        do_chunks(node_idx_h, node_tab_h, wid, 0)

    @pl.when(wid >= 16)
    def _():
        do_chunks(edge_idx_h, edge_tab_h, wid - 16, R)


def _make_sc_call():
    mesh = plsc.VectorSubcoreMesh(core_axis_name="c", subcore_axis_name="s")
    return pl.kernel(
        _sc_gather,
        out_type=jax.ShapeDtypeStruct((2 * R, D), jnp.float32),
        mesh=mesh,
        compiler_params=pltpu.CompilerParams(use_tc_tiling_on_sc=False),
        scratch_types=[
            pltpu.VMEM((CHUNK,), jnp.int32),
            pltpu.VMEM((CHUNK, D), jnp.float32),
            pltpu.SemaphoreType.DMA,
        ],
    )


def _mm_body(xt_ref, wt_ref, pet_ref, out_ref):
    prod = jnp.dot(wt_ref[...], xt_ref[...], preferred_element_type=jnp.float32)
    for j in range(MM_LS):
        out_ref[j] = prod[:, j * B:(j + 1) * B] + pet_ref[j]


def _mm_call(xt, wt, pet):
    return pl.pallas_call(
        _mm_body,
        grid=(L // MM_LS,),
        in_specs=[
            pl.BlockSpec((K, MM_LS * B), lambda i: (0, i)),
            pl.BlockSpec((D, K), lambda i: (0, 0)),
            pl.BlockSpec((MM_LS, D, 1), lambda i: (i, 0, 0)),
        ],
        out_specs=pl.BlockSpec((MM_LS, D, B), lambda i: (2 * L // MM_LS + i, 0, 0)),
        out_shape=jax.ShapeDtypeStruct((3 * L, D, B), jnp.float32),
    )(xt, wt, pet)


def _fix_body(alias_ref, tmp_ref, sel_ref, pet_ref, out_ref):
    del alias_ref
    b_i = lax.broadcasted_iota(jnp.int32, (D, B), 1)
    for j in range(FIX_LS):
        t = tmp_ref[pl.ds(j * CHUNK, CHUNK), :]
        h = lax.dot_general(t, sel_ref[...], (((0,), (0,)), ((), ())),
                            preferred_element_type=jnp.float32)
        out = jnp.where(b_i % 2 == 0, h[:D, :], h[D:, :])
        out_ref[j] = out + pet_ref[j]


def _fix_call(out_a, tmp, sel, pet):
    n_pe_blocks = L // FIX_LS
    return pl.pallas_call(
        _fix_body,
        grid=(2 * L // FIX_LS,),
        in_specs=[
            pl.BlockSpec(memory_space=pl.ANY),
            pl.BlockSpec((FIX_LS * CHUNK, 2 * D), lambda i: (i, 0)),
            pl.BlockSpec((CHUNK, B), lambda i: (0, 0)),
            pl.BlockSpec((FIX_LS, D, 1),
                         lambda i: (lax.rem(i, n_pe_blocks), 0, 0)),
        ],
        out_specs=pl.BlockSpec((FIX_LS, D, B), lambda i: (i, 0, 0)),
        out_shape=jax.ShapeDtypeStruct((3 * L, D, B), jnp.float32),
        input_output_aliases={0: 0},
    )(out_a, tmp, sel, pet)


def kernel(node_idx, edge_idx, node_val_mat, node_embed_table,
           edge_embed_table, val_tok_embed):
    pet = jnp.asarray(_PE3)
    sel = jnp.asarray(_SEL)
    xt = jnp.transpose(node_val_mat)
    wt = jnp.transpose(val_tok_embed)
    nt_lin = lax.optimization_barrier(node_embed_table.reshape(-1))
    tmp = _make_sc_call()(
        node_idx.reshape(-1).astype(jnp.int32),
        edge_idx.reshape(-1).astype(jnp.int32),
        nt_lin.reshape(100000, D),
        edge_embed_table,
    ).reshape(R, 2 * D)
    out_a = _mm_call(xt, wt, pet)
    out = _fix_call(out_a, tmp, sel, pet)
    return jnp.transpose(out.reshape(3 * L, D, B), (0, 2, 1))


# speedup vs baseline: 1.1512x; 1.0062x over previous
import jax
import jax.numpy as jnp
import numpy as np
from jax import lax
from jax.experimental import pallas as pl
from jax.experimental.pallas import tpu as pltpu
from jax.experimental.pallas import tpu_sc as plsc

L, B, D = 200, 256, 64
R = L * B
K = 1000
CHUNK = 128
NODE_CHUNKS = R // CHUNK
CPW = NODE_CHUNKS // 16
MM_LS = 20
FIX_LS = 25

_SEL = np.zeros((CHUNK, B), dtype=np.float32)
_SEL[np.arange(B) // 2, np.arange(B)] = 1.0


def _pe_np():
    pos = np.arange(L, dtype=np.float32)[:, None]
    div = np.exp(np.arange(0, D, 2, dtype=np.float32) * (-np.log(10000.0) / D))
    pe = np.zeros((L, D), dtype=np.float32)
    pe[:, 0::2] = np.sin(pos * div)
    pe[:, 1::2] = np.cos(pos * div)
    return pe


_PE3 = _pe_np()[:, :, None]


def _sc_gather(node_idx_h, edge_idx_h, node_tab_h, edge_tab_h,
               out_h, idx_v, rows_v, sem):
    cid = lax.axis_index("c")
    sid = lax.axis_index("s")
    wid = sid * 2 + cid

    def do_chunks(idx_h, tab_h, w, out_row0):
        def body(k, _):
            c = w * CPW + k
            pltpu.sync_copy(idx_h.at[pl.ds(c * CHUNK, CHUNK)], idx_v)
            pltpu.async_copy(tab_h.at[idx_v], rows_v, sem).wait()
            pltpu.sync_copy(
                rows_v,
                out_h.at[pl.ds(out_row0 + c * CHUNK, CHUNK)],
            )
            return 0

        lax.fori_loop(0, CPW, body, 0)

    @pl.when(wid < 16)
    def _():
        do_chunks(node_idx_h, node_tab_h, wid, 0)

    @pl.when(wid >= 16)
    def _():
        do_chunks(edge_idx_h, edge_tab_h, wid - 16, R)


def _make_sc_call():
    mesh = plsc.VectorSubcoreMesh(core_axis_name="c", subcore_axis_name="s")
    return pl.kernel(
        _sc_gather,
        out_type=jax.ShapeDtypeStruct((2 * R, D), jnp.float32),
        mesh=mesh,
        compiler_params=pltpu.CompilerParams(use_tc_tiling_on_sc=False),
        scratch_types=[
            pltpu.VMEM((CHUNK,), jnp.int32),
            pltpu.VMEM((CHUNK, D), jnp.float32),
            pltpu.SemaphoreType.DMA,
        ],
    )


def _mm_body(xt_ref, wt_ref, pet_ref, out_ref):
    prod = jnp.dot(wt_ref[...], xt_ref[...], preferred_element_type=jnp.float32)
    for j in range(MM_LS):
        out_ref[j] = prod[:, j * B:(j + 1) * B] + pet_ref[j]


def _mm_call(xt, wt, pet):
    return pl.pallas_call(
        _mm_body,
        grid=(L // MM_LS,),
        in_specs=[
            pl.BlockSpec((K, MM_LS * B), lambda i: (0, i)),
            pl.BlockSpec((D, K), lambda i: (0, 0)),
            pl.BlockSpec((MM_LS, D, 1), lambda i: (i, 0, 0)),
        ],
        out_specs=pl.BlockSpec((MM_LS, D, B), lambda i: (2 * L // MM_LS + i, 0, 0)),
        out_shape=jax.ShapeDtypeStruct((3 * L, D, B), jnp.float32),
    )(xt, wt, pet)


def _fix_body(alias_ref, tmp_ref, sel_ref, pet_ref, out_ref):
    del alias_ref
    b_i = lax.broadcasted_iota(jnp.int32, (D, B), 1)
    for j in range(FIX_LS):
        t = tmp_ref[pl.ds(j * CHUNK, CHUNK), :]
        h = lax.dot_general(t, sel_ref[...], (((0,), (0,)), ((), ())),
                            preferred_element_type=jnp.float32)
        out = jnp.where(b_i % 2 == 0, h[:D, :], h[D:, :])
        out_ref[j] = out + pet_ref[j]


def _fix_call(out_a, tmp, sel, pet):
    n_pe_blocks = L // FIX_LS
    return pl.pallas_call(
        _fix_body,
        grid=(2 * L // FIX_LS,),
        in_specs=[
            pl.BlockSpec(memory_space=pl.ANY),
            pl.BlockSpec((FIX_LS * CHUNK, 2 * D), lambda i: (i, 0)),
            pl.BlockSpec((CHUNK, B), lambda i: (0, 0)),
            pl.BlockSpec((FIX_LS, D, 1),
                         lambda i: (lax.rem(i, n_pe_blocks), 0, 0)),
        ],
        out_specs=pl.BlockSpec((FIX_LS, D, B), lambda i: (i, 0, 0)),
        out_shape=jax.ShapeDtypeStruct((3 * L, D, B), jnp.float32),
        input_output_aliases={0: 0},
    )(out_a, tmp, sel, pet)


def kernel(node_idx, edge_idx, node_val_mat, node_embed_table,
           edge_embed_table, val_tok_embed):
    pet = jnp.asarray(_PE3)
    sel = jnp.asarray(_SEL)
    xt = jnp.transpose(node_val_mat)
    wt = jnp.transpose(val_tok_embed)
    nt_lin = lax.optimization_barrier(node_embed_table.reshape(-1))
    tmp = _make_sc_call()(
        node_idx.reshape(-1).astype(jnp.int32),
        edge_idx.reshape(-1).astype(jnp.int32),
        nt_lin.reshape(100000, D),
        edge_embed_table,
    ).reshape(R, 2 * D)
    out_a = _mm_call(xt, wt, pet)
    out = _fix_call(out_a, tmp, sel, pet)
    return jnp.transpose(out.reshape(3 * L, D, B), (0, 2, 1))
